# baseline (speedup 1.0000x reference)
import functools

import jax
import jax.numpy as jnp
from jax import lax
from jax.experimental import pallas as pl
from jax.experimental.pallas import tpu as pltpu
from jax.experimental.pallas import tpu_sc as plsc

F32 = jnp.float32
DH = 128
NC, NS = 2, 16
CH = 128

N_SIZES = (50000, 50000, 10000, 10000)
E_SIZES = (200000, 200000, 100000, 100000)
T_SRCS = (0, 1, 2, 3)
T_TGTS = (1, 0, 3, 2)
N_LAYERS = 5

R_BIG, R_SMALL = 12544, 5120
N_PADS = (2 * 2 * R_BIG, 2 * 2 * R_BIG, 2 * R_SMALL, 2 * R_SMALL)
R_SIZES = (R_BIG, R_BIG, R_SMALL, R_SMALL)
N_PASSES = (2, 2, 1, 1)
E_PADS = (200704, 200704, 100352, 100352)


def _mm_body(x_ref, w_ref, b_ref, o_ref, *, in_relu):
    x = x_ref[...]
    if in_relu:
        x = jnp.maximum(x, 0.0)
    acc = jnp.dot(x, w_ref[...], preferred_element_type=F32) + b_ref[...]
    o_ref[...] = jnp.maximum(acc, 0.0)


def _msg_mm(x, w, b, in_relu):
    n, k = x.shape
    blk = 512
    return pl.pallas_call(
        functools.partial(_mm_body, in_relu=in_relu),
        grid=(n // blk,),
        in_specs=[
            pl.BlockSpec((blk, k), lambda i: (i, 0)),
            pl.BlockSpec((k, DH), lambda i: (0, 0)),
            pl.BlockSpec((1, DH), lambda i: (0, 0)),
        ],
        out_specs=pl.BlockSpec((blk, DH), lambda i: (i, 0)),
        out_shape=jax.ShapeDtypeStruct((n, DH), F32),
    )(x, w, b.reshape(1, DH))


def _enc2_body(xa_ref, xb_ref, wa_ref, wb_ref, b_ref, o_ref):
    acc = jnp.dot(xa_ref[...], wa_ref[...], preferred_element_type=F32)
    acc += jnp.dot(xb_ref[...], wb_ref[...], preferred_element_type=F32)
    o_ref[...] = acc + b_ref[...]


def _enc2(xa, xb, wa, wb, b):
    n, ka = xa.shape
    kb = xb.shape[1]
    blk = 512
    return pl.pallas_call(
        _enc2_body,
        grid=(n // blk,),
        in_specs=[
            pl.BlockSpec((blk, ka), lambda i: (i, 0)),
            pl.BlockSpec((blk, kb), lambda i: (i, 0)),
            pl.BlockSpec((ka, DH), lambda i: (0, 0)),
            pl.BlockSpec((kb, DH), lambda i: (0, 0)),
            pl.BlockSpec((1, DH), lambda i: (0, 0)),
        ],
        out_specs=pl.BlockSpec((blk, DH), lambda i: (i, 0)),
        out_shape=jax.ShapeDtypeStruct((n, DH), F32),
    )(xa, xb, wa, wb, b.reshape(1, DH))


def _emb_body(x_ref, e_ref, o_ref):
    o_ref[...] = x_ref[...] * e_ref[...]


def _enc_emb(x, emb):
    n = x.shape[0]
    blk = 512
    return pl.pallas_call(
        _emb_body,
        grid=(n // blk,),
        in_specs=[
            pl.BlockSpec((blk, 1), lambda i: (i, 0)),
            pl.BlockSpec((1, DH), lambda i: (0, 0)),
        ],
        out_specs=pl.BlockSpec((blk, DH), lambda i: (i, 0)),
        out_shape=jax.ShapeDtypeStruct((n, DH), F32),
    )(x, emb.reshape(1, DH))


def _relu_body(x_ref, o_ref):
    o_ref[...] = jnp.maximum(x_ref[...], 0.0)


def _relu_slice(x, n_out):
    blk = 400
    return pl.pallas_call(
        _relu_body,
        grid=(n_out // blk,),
        in_specs=[pl.BlockSpec((blk, DH), lambda i: (i, 0))],
        out_specs=pl.BlockSpec((blk, DH), lambda i: (i, 0)),
        out_shape=jax.ShapeDtypeStruct((n_out, DH), F32),
    )(x)


@functools.lru_cache(maxsize=None)
def _make_sc_edge(e_pad, n_tab, n_out, r, n_pass):
    per_sub = e_pad // NS
    n_chunks = per_sub // CH
    acc_rows = r + 16
    zone = acc_rows // NS
    out_ps = r // NS
    mesh = plsc.VectorSubcoreMesh(core_axis_name="c", subcore_axis_name="s")

    @functools.partial(
        pl.kernel,
        out_type=jax.ShapeDtypeStruct((n_out, DH), F32),
        mesh=mesh,
        scratch_types=[
            pltpu.VMEM((CH,), jnp.int32),
            pltpu.VMEM((CH,), jnp.int32),
            pltpu.VMEM((1, CH), jnp.int32),
            pltpu.VMEM((CH,), F32),
            pltpu.VMEM((CH, DH), F32),
            pltpu.VMEM((32, DH), F32),
            pltpu.VMEM_SHARED((acc_rows, DH), F32),
        ],
    )
    def sc_edge(msg, src, tgt, w, out, srcb, tgtb, lclb, wb, rows, zbuf, acc):
        c = lax.axis_index("c")
        s = lax.axis_index("s")
        e_base = s * per_sub

        z16 = jnp.zeros((16,), F32)

        @pl.loop(0, 32)
        def _(rr):
            for cg in range(DH // 16):
                zbuf[rr, pl.ds(cg * 16, 16)] = z16

        for p in range(n_pass):
            base = (2 * p) * r + c * r

            z0 = s * zone
            off = 0
            sizes = [32] * (zone // 32) + ([zone % 32] if zone % 32 else [])
            for ncopy in sizes:
                pltpu.sync_copy(zbuf.at[pl.ds(0, ncopy)],
                                acc.at[pl.ds(z0 + off, ncopy)])
                off += ncopy
            plsc.subcore_barrier()

            @pl.loop(0, n_chunks)
            def _(g):
                e0 = e_base + g * CH
                pltpu.sync_copy(src.at[pl.ds(e0, CH)], srcb)
                pltpu.sync_copy(tgt.at[pl.ds(e0, CH)], tgtb)
                pltpu.sync_copy(w.at[pl.ds(e0, CH)], wb)
                pltpu.sync_copy(msg.at[srcb], rows)

                @pl.loop(0, CH // 16)
                def _(gg):
                    sl = pl.ds(gg * 16, 16)
                    lcl = tgtb[sl] - base
                    inr = (lcl >= 0) & (lcl < r)
                    wv = jnp.where(inr, wb[sl], 0.0)
                    lclb[0, sl] = jnp.where(inr, lcl, r)
                    for e in range(16):
                        ws = wv[e]
                        ri = gg * 16 + e
                        for cg in range(DH // 16):
                            sl2 = pl.ds(cg * 16, 16)
                            rows[ri, sl2] = rows[ri, sl2] * ws

                pltpu.sync_copy(rows, acc.at[lclb.at[0]], add=True)

            plsc.subcore_barrier()
            o0 = s * out_ps
            pltpu.sync_copy(acc.at[pl.ds(o0, out_ps)],
                            out.at[pl.ds(base + o0, out_ps)])
            plsc.subcore_barrier()

    return sc_edge


def _pad_rows(x, n_pad):
    return jnp.pad(x, ((0, n_pad - x.shape[0]), (0, 0)))


def _pad_1d(x, e_pad):
    return jnp.pad(x, (0, e_pad - x.shape[0]))


def kernel(x_cl, x_cc, x_al, x_ac, x_pt, x_ft, es0, es1, es2, es3,
           w0, w1, w2, w3, W_cl, b_cl, W_cc, b_cc, W_ac, b_ac,
           emb_pt, emb_ft, W_mpnn, b_mpnn):
    ess = (es0, es1, es2, es3)
    ws = (w0, w1, w2, w3)

    srcs, tgts, wts = [], [], []
    for j in range(4):
        srcs.append(_pad_1d(ess[j][0], E_PADS[j]))
        tgts.append(_pad_1d(ess[j][1], E_PADS[j]))
        wts.append(_pad_1d(ws[j], E_PADS[j]))

    xs = [
        _enc2(_pad_rows(x_cl, N_PADS[0]), _pad_rows(x_cc, N_PADS[0]),
              W_cl, W_cc, b_cl + b_cc),
        _enc2(_pad_rows(x_al, N_PADS[1]), _pad_rows(x_ac, N_PADS[1]),
              W_ac, W_ac, 2.0 * b_ac),
        _enc_emb(_pad_rows(x_pt, N_PADS[2]), emb_pt),
        _enc_emb(_pad_rows(x_ft, N_PADS[3]), emb_ft),
    ]

    for i in range(N_LAYERS):
        new_xs = [None] * 4
        for j in range(4):
            ts, tt = T_SRCS[j], T_TGTS[j]
            msg = _msg_mm(xs[ts], W_mpnn[i, j], b_mpnn[i, j], in_relu=(i > 0))
            sc = _make_sc_edge(E_PADS[j], N_PADS[ts], N_PADS[tt],
                               R_SIZES[tt], N_PASSES[tt])
            new_xs[tt] = sc(msg, srcs[j], tgts[j], wts[j])
        xs = new_xs

    return tuple(_relu_slice(xs[t], N_SIZES[t]) for t in range(4))


# speedup vs baseline: 1.7473x; 1.7473x over previous
import dataclasses
import functools

import jax
import jax.numpy as jnp
from jax import lax
from jax.experimental import pallas as pl
from jax.experimental.pallas import tpu as pltpu
from jax.experimental.pallas import tpu_sc as plsc

F32 = jnp.float32
I32 = jnp.int32
DH = 128
NC, NS = 2, 16
NW = NC * NS
CH = 128

N_SIZES = (50000, 50000, 10000, 10000)
T_SRCS = (0, 1, 2, 3)
T_TGTS = (1, 0, 3, 2)
N_LAYERS = 5

R_BIG, R_SMALL = 12544, 5120
N_PADS = (4 * R_BIG, 4 * R_BIG, 2 * R_SMALL, 2 * R_SMALL)
R_SIZES = (R_BIG, R_BIG, R_SMALL, R_SMALL)
NBS = (4, 4, 2, 2)
E_PADS = (200704, 200704, 100352, 100352)
SEGS = (6272, 6272, 3200, 3200)


def _mm_body(x_ref, w_ref, b_ref, o_ref, *, in_relu):
    x = x_ref[...]
    if in_relu:
        x = jnp.maximum(x, 0.0)
    acc = jnp.dot(x, w_ref[...], preferred_element_type=F32) + b_ref[...]
    o_ref[...] = jnp.maximum(acc, 0.0)


def _msg_mm(x, w, b, in_relu):
    n, k = x.shape
    blk = 512
    return pl.pallas_call(
        functools.partial(_mm_body, in_relu=in_relu),
        grid=(n // blk,),
        in_specs=[
            pl.BlockSpec((blk, k), lambda i: (i, 0)),
            pl.BlockSpec((k, DH), lambda i: (0, 0)),
            pl.BlockSpec((1, DH), lambda i: (0, 0)),
        ],
        out_specs=pl.BlockSpec((blk, DH), lambda i: (i, 0)),
        out_shape=jax.ShapeDtypeStruct((n, DH), F32),
    )(x, w, b.reshape(1, DH))


def _enc2_body(xa_ref, xb_ref, wa_ref, wb_ref, b_ref, o_ref):
    acc = jnp.dot(xa_ref[...], wa_ref[...], preferred_element_type=F32)
    acc += jnp.dot(xb_ref[...], wb_ref[...], preferred_element_type=F32)
    o_ref[...] = acc + b_ref[...]


def _enc2(xa, xb, wa, wb, b):
    n, ka = xa.shape
    kb = xb.shape[1]
    blk = 512
    return pl.pallas_call(
        _enc2_body,
        grid=(n // blk,),
        in_specs=[
            pl.BlockSpec((blk, ka), lambda i: (i, 0)),
            pl.BlockSpec((blk, kb), lambda i: (i, 0)),
            pl.BlockSpec((ka, DH), lambda i: (0, 0)),
            pl.BlockSpec((kb, DH), lambda i: (0, 0)),
            pl.BlockSpec((1, DH), lambda i: (0, 0)),
        ],
        out_specs=pl.BlockSpec((blk, DH), lambda i: (i, 0)),
        out_shape=jax.ShapeDtypeStruct((n, DH), F32),
    )(xa, xb, wa, wb, b.reshape(1, DH))


def _emb_body(x_ref, e_ref, o_ref):
    o_ref[...] = x_ref[...] * e_ref[...]


def _enc_emb(x, emb):
    n = x.shape[0]
    blk = 512
    return pl.pallas_call(
        _emb_body,
        grid=(n // blk,),
        in_specs=[
            pl.BlockSpec((blk, 1), lambda i: (i, 0)),
            pl.BlockSpec((1, DH), lambda i: (0, 0)),
        ],
        out_specs=pl.BlockSpec((blk, DH), lambda i: (i, 0)),
        out_shape=jax.ShapeDtypeStruct((n, DH), F32),
    )(x, emb.reshape(1, DH))


def _relu_body(x_ref, o_ref):
    o_ref[...] = jnp.maximum(x_ref[...], 0.0)


def _relu_slice(x, n_out):
    blk = 400
    return pl.pallas_call(
        _relu_body,
        grid=(n_out // blk,),
        in_specs=[pl.BlockSpec((blk, DH), lambda i: (i, 0))],
        out_specs=pl.BlockSpec((blk, DH), lambda i: (i, 0)),
        out_shape=jax.ShapeDtypeStruct((n_out, DH), F32),
    )(x)


def _sc_mesh():
    return plsc.VectorSubcoreMesh(core_axis_name="c", subcore_axis_name="s")


def _sc_params():
    cp = pltpu.CompilerParams()
    if "needs_layout_passes" in pltpu.CompilerParams.__dataclass_fields__:
        cp = dataclasses.replace(cp, needs_layout_passes=False)
    return cp


@functools.lru_cache(maxsize=None)
def _make_sc_bucket(e_pad, nb, r, seg):
    per_w = e_pad // NW
    full_chunks = per_w // CH
    tail16 = (per_w - full_chunks * CH) // 16
    stride = seg + 16

    @functools.partial(
        pl.kernel,
        out_type=(
            jax.ShapeDtypeStruct((NW * nb * seg,), I32),
            jax.ShapeDtypeStruct((NW * nb * seg,), I32),
            jax.ShapeDtypeStruct((NW * nb * seg,), F32),
            jax.ShapeDtypeStruct((NW * 16,), I32),
        ),
        mesh=_sc_mesh(),
        scratch_types=[
            pltpu.VMEM((CH,), I32),
            pltpu.VMEM((CH,), I32),
            pltpu.VMEM((CH,), F32),
            pltpu.VMEM((nb * stride,), I32),
            pltpu.VMEM((nb * stride,), I32),
            pltpu.VMEM((nb * stride,), F32),
            pltpu.VMEM((16,), I32),
        ],
        compiler_params=_sc_params(),
    )
    def bucket_k(src, tgt, w, osrc, otgt, ow, ocnt,
                 srcb, tgtb, wb, ssrc, stgt, sw, scnt):
        c = lax.axis_index("c")
        s = lax.axis_index("s")
        wid = c * NS + s
        ebase = wid * per_w
        lanes = lax.iota(I32, 16)

        def do_chunk(e0, n16, pos):
            n = n16 * 16
            pltpu.sync_copy(src.at[pl.ds(e0, n)], srcb.at[pl.ds(0, n)])
            pltpu.sync_copy(tgt.at[pl.ds(e0, n)], tgtb.at[pl.ds(0, n)])
            pltpu.sync_copy(w.at[pl.ds(e0, n)], wb.at[pl.ds(0, n)])
            for gg in range(n16):
                sl = pl.ds(gg * 16, 16)
                tv = tgtb[sl]
                sv = srcb[sl]
                wv = wb[sl]
                bk = jnp.where(tv >= r, 1, 0)
                for m in range(2, nb):
                    bk = bk + jnp.where(tv >= m * r, 1, 0)
                poslane = jnp.zeros((16,), I32)
                for b in range(nb):
                    mb = bk == b
                    pc = plsc.cumsum(jnp.where(mb, 1, 0))
                    poslane = jnp.where(mb, b * stride + pos[b] + pc - 1,
                                        poslane)
                    pos = pos + jnp.where(lanes == b, pc[15], 0)
                plsc.store_scatter(ssrc, [poslane], sv)
                plsc.store_scatter(stgt, [poslane], tv)
                plsc.store_scatter(sw, [poslane], wv)
            return pos

        def body(g, pos):
            return do_chunk(ebase + g * CH, CH // 16, pos)

        pos = lax.fori_loop(0, full_chunks, body, jnp.zeros((16,), I32))
        if tail16:
            pos = do_chunk(ebase + full_chunks * CH, tail16, pos)

        scnt[pl.ds(0, 16)] = pos
        for b in range(nb):
            o0 = (wid * nb + b) * seg
            sb = pl.ds(b * stride, seg)
            ob = pl.ds(o0, seg)
            pltpu.sync_copy(ssrc.at[sb], osrc.at[ob])
            pltpu.sync_copy(stgt.at[sb], otgt.at[ob])
            pltpu.sync_copy(sw.at[sb], ow.at[ob])
        pltpu.sync_copy(scnt, ocnt.at[pl.ds(wid * 16, 16)])

    return bucket_k


@functools.lru_cache(maxsize=None)
def _make_sc_edge(n_tab, n_out, r, nb, seg):
    n_pass = nb // 2
    cap = seg // CH
    acc_rows = r + 16
    zone = acc_rows // NS
    out_ps = r // NS
    zsizes = [32] * (zone // 32) + ([zone % 32] if zone % 32 else [])

    @functools.partial(
        pl.kernel,
        out_type=jax.ShapeDtypeStruct((n_out, DH), F32),
        mesh=_sc_mesh(),
        scratch_types=[
            pltpu.VMEM((CH,), I32),
            pltpu.VMEM((CH,), I32),
            pltpu.VMEM((1, CH), I32),
            pltpu.VMEM((CH,), F32),
            pltpu.VMEM((CH, DH), F32),
            pltpu.VMEM((32, DH), F32),
            pltpu.VMEM((NW * 16,), I32),
            pltpu.VMEM_SHARED((acc_rows, DH), F32),
        ],
        compiler_params=_sc_params(),
    )
    def edge_k(msg, sseg, tseg, wseg, cnts, out,
               srcb, tgtb, lclb, wb, rows, zbuf, cntb, acc):
        c = lax.axis_index("c")
        s = lax.axis_index("s")
        lanes = lax.iota(I32, 16)
        pltpu.sync_copy(cnts, cntb)

        z16 = jnp.zeros((16,), F32)

        @pl.loop(0, 32)
        def _(rr):
            for cg in range(DH // 16):
                zbuf[rr, pl.ds(cg * 16, 16)] = z16

        for p in range(n_pass):
            idx = 2 * p + c
            base = idx * r

            z0 = s * zone
            off = 0
            for ncopy in zsizes:
                pltpu.sync_copy(zbuf.at[pl.ds(0, ncopy)],
                                acc.at[pl.ds(z0 + off, ncopy)])
                off += ncopy
            plsc.subcore_barrier()

            for segi in range(2):
                sgm = 2 * s + segi
                cv = cntb[pl.ds(sgm * 16, 16)]
                count = jnp.sum(jnp.where(lanes == idx, cv, 0))
                segbase = (sgm * nb + idx) * seg

                @pl.loop(0, cap)
                def _(g):
                    @pl.when(g * CH < count)
                    def _():
                        e0 = segbase + g * CH
                        pltpu.sync_copy(sseg.at[pl.ds(e0, CH)], srcb)
                        pltpu.sync_copy(tseg.at[pl.ds(e0, CH)], tgtb)
                        pltpu.sync_copy(wseg.at[pl.ds(e0, CH)], wb)
                        for gg in range(CH // 16):
                            sl = pl.ds(gg * 16, 16)
                            srcb[sl] = jnp.minimum(
                                jnp.maximum(srcb[sl], 0), n_tab - 1)
                        pltpu.sync_copy(msg.at[srcb], rows)

                        @pl.loop(0, CH // 16)
                        def _(gg):
                            sl = pl.ds(gg * 16, 16)
                            eid = g * CH + gg * 16 + lanes
                            lcl = tgtb[sl] - base
                            inr = (eid < count) & (lcl >= 0) & (lcl < r)
                            wv = jnp.where(inr, wb[sl], 0.0)
                            lclb[0, sl] = jnp.where(inr, lcl, r)
                            for e in range(16):
                                ws = wv[e]
                                ri = gg * 16 + e
                                for cg in range(DH // 16):
                                    sl2 = pl.ds(cg * 16, 16)
                                    rows[ri, sl2] = rows[ri, sl2] * ws

                        pltpu.sync_copy(rows, acc.at[lclb.at[0]], add=True)

            plsc.subcore_barrier()
            o0 = s * out_ps
            pltpu.sync_copy(acc.at[pl.ds(o0, out_ps)],
                            out.at[pl.ds(base + o0, out_ps)])
            plsc.subcore_barrier()

    return edge_k


def _pad_rows(x, n_pad):
    return jnp.pad(x, ((0, n_pad - x.shape[0]), (0, 0)))


def _pad_1d(x, e_pad):
    return jnp.pad(x, (0, e_pad - x.shape[0]))


def kernel(x_cl, x_cc, x_al, x_ac, x_pt, x_ft, es0, es1, es2, es3,
           w0, w1, w2, w3, W_cl, b_cl, W_cc, b_cc, W_ac, b_ac,
           emb_pt, emb_ft, W_mpnn, b_mpnn):
    ess = (es0, es1, es2, es3)
    ws = (w0, w1, w2, w3)

    segd = []
    for j in range(4):
        tt = T_TGTS[j]
        bucket = _make_sc_bucket(E_PADS[j], NBS[tt], R_SIZES[tt], SEGS[tt])
        segd.append(bucket(_pad_1d(ess[j][0], E_PADS[j]),
                           _pad_1d(ess[j][1], E_PADS[j]),
                           _pad_1d(ws[j], E_PADS[j])))

    xs = [
        _enc2(_pad_rows(x_cl, N_PADS[0]), _pad_rows(x_cc, N_PADS[0]),
              W_cl, W_cc, b_cl + b_cc),
        _enc2(_pad_rows(x_al, N_PADS[1]), _pad_rows(x_ac, N_PADS[1]),
              W_ac, W_ac, 2.0 * b_ac),
        _enc_emb(_pad_rows(x_pt, N_PADS[2]), emb_pt),
        _enc_emb(_pad_rows(x_ft, N_PADS[3]), emb_ft),
    ]

    for i in range(N_LAYERS):
        new_xs = [None] * 4
        for j in range(4):
            ts, tt = T_SRCS[j], T_TGTS[j]
            msg = _msg_mm(xs[ts], W_mpnn[i, j], b_mpnn[i, j], in_relu=(i > 0))
            edge = _make_sc_edge(N_PADS[ts], N_PADS[tt], R_SIZES[tt],
                                 NBS[tt], SEGS[tt])
            sseg, tseg, wseg, cnts = segd[j]
            new_xs[tt] = edge(msg, sseg, tseg, wseg, cnts)
        xs = new_xs

    return tuple(_relu_slice(xs[t], N_SIZES[t]) for t in range(4))
